# Initial kernel scaffold; baseline (speedup 1.0000x reference)
import jax
import jax.numpy as jnp
from jax.experimental import pallas as pl


def kernel(x_checkin, x_poi, ei_seq, ei_visits, ei_visited, ei_spatial, Wpc, bpc, Wpp, bpp, prelu_a, l1_seq_W, l1_seq_b, l1_vis_Wl, l1_vis_bl, l1_vis_Wr, l1_vtd_Wl, l1_vtd_bl, l1_vtd_Wr, l1_sp_W, l1_sp_b, l2_seq_W, l2_seq_b, l2_vis_Wl, l2_vis_bl, l2_vis_Wr, l2_vtd_Wl, l2_vtd_bl, l2_vtd_Wr, l2_sp_W, l2_sp_b):
    raise NotImplementedError("write your pallas kernel here")


# speedup vs baseline: 1.8531x; 1.8531x over previous
import functools

import jax
import jax.numpy as jnp
from jax.experimental import pallas as pl

N_CHECKIN = 100000
N_POI = 20000
HIDDEN = 128
BLK = 1000


def _transform_body(x_ref, W_ref, b_ref, deg_ref, h_ref, hsc_ref):
    h = jnp.dot(x_ref[:], W_ref[:], preferred_element_type=jnp.float32,
                precision=jax.lax.Precision.HIGHEST) + b_ref[:]
    deg = deg_ref[:]
    dis = jnp.where(deg > 0.0, jax.lax.rsqrt(jnp.maximum(deg, 1e-12)), 0.0)
    h_ref[:] = h
    hsc_ref[:] = dis * h


def _transform(x, W, b, deg, n):
    grid = (n // BLK,)
    row = pl.BlockSpec((BLK, HIDDEN), lambda i: (i, 0))
    return pl.pallas_call(
        _transform_body,
        grid=grid,
        in_specs=[
            row,
            pl.BlockSpec((HIDDEN, HIDDEN), lambda i: (0, 0)),
            pl.BlockSpec((1, HIDDEN), lambda i: (0, 0)),
            pl.BlockSpec((BLK, 1), lambda i: (i, 0)),
        ],
        out_specs=[row, row],
        out_shape=[jax.ShapeDtypeStruct((n, HIDDEN), jnp.float32)] * 2,
    )(x, W, b.reshape(1, HIDDEN), deg)


def _combine_body(agg1_ref, agg2_ref, h_ref, deg_ref, cnt_ref,
                  W1_ref, W2_ref, W3_ref, b1_ref, b2_ref, pa_ref,
                  c_ref, csc_ref, *, with_prelu, with_scaled):
    deg = deg_ref[:]
    dis = jnp.where(deg > 0.0, jax.lax.rsqrt(jnp.maximum(deg, 1e-12)), 0.0)
    invc = 1.0 / jnp.maximum(cnt_ref[:], 1.0)
    hi = jax.lax.Precision.HIGHEST
    t = dis * jnp.dot(agg1_ref[:], W1_ref[:],
                      preferred_element_type=jnp.float32, precision=hi)
    t = t + b1_ref[:] + b2_ref[:]
    t = t + jnp.dot(invc * agg2_ref[:], W2_ref[:],
                    preferred_element_type=jnp.float32, precision=hi)
    t = t + jnp.dot(h_ref[:], W3_ref[:],
                    preferred_element_type=jnp.float32, precision=hi)
    if with_prelu:
        pa = pa_ref[0, 0]
        t = jnp.where(t >= 0.0, t, pa * t)
    c_ref[:] = t
    if with_scaled:
        csc_ref[:] = dis * t


def _combine(agg1, agg2, h, deg, cnt, W1, W2, W3, b1, b2, pa, n,
             with_prelu, with_scaled):
    grid = (n // BLK,)
    row = pl.BlockSpec((BLK, HIDDEN), lambda i: (i, 0))
    wspec = pl.BlockSpec((HIDDEN, HIDDEN), lambda i: (0, 0))
    bspec = pl.BlockSpec((1, HIDDEN), lambda i: (0, 0))
    col = pl.BlockSpec((BLK, 1), lambda i: (i, 0))
    nout = 2 if with_scaled else 1
    body = functools.partial(_combine_body, with_prelu=with_prelu,
                             with_scaled=with_scaled)
    if not with_scaled:
        def body2(a1, a2, hh, dg, ct, w1, w2, w3, bb1, bb2, paa, c):
            body(a1, a2, hh, dg, ct, w1, w2, w3, bb1, bb2, paa, c, None)
        fn = body2
    else:
        fn = body
    out = pl.pallas_call(
        fn,
        grid=grid,
        in_specs=[row, row, row, col, col, wspec, wspec, wspec, bspec, bspec,
                  pl.BlockSpec((1, 1), lambda i: (0, 0))],
        out_specs=[row] * nout,
        out_shape=[jax.ShapeDtypeStruct((n, HIDDEN), jnp.float32)] * nout,
    )(agg1, agg2, h, deg, cnt, W1, W2, W3,
      b1.reshape(1, HIDDEN), b2.reshape(1, HIDDEN), pa.reshape(1, 1))
    return out if with_scaled else (out[0], None)


def _segsum(table, src, dst, n_dst):
    return jax.ops.segment_sum(table[src], dst, num_segments=n_dst)


def _counts(dst, n_dst):
    ones = jnp.ones(dst.shape[0], jnp.float32)
    return jax.ops.segment_sum(ones, dst, num_segments=n_dst).reshape(n_dst, 1)


def kernel(x_checkin, x_poi, ei_seq, ei_visits, ei_visited, ei_spatial,
           Wpc, bpc, Wpp, bpp, prelu_a,
           l1_seq_W, l1_seq_b, l1_vis_Wl, l1_vis_bl, l1_vis_Wr,
           l1_vtd_Wl, l1_vtd_bl, l1_vtd_Wr, l1_sp_W, l1_sp_b,
           l2_seq_W, l2_seq_b, l2_vis_Wl, l2_vis_bl, l2_vis_Wr,
           l2_vtd_Wl, l2_vtd_bl, l2_vtd_Wr, l2_sp_W, l2_sp_b):
    pa = jnp.asarray(prelu_a, jnp.float32)
    deg_seq = _counts(ei_seq[1], N_CHECKIN)
    cnt_vtd = _counts(ei_visited[1], N_CHECKIN)
    cnt_vis = _counts(ei_visits[1], N_POI)
    deg_sp = _counts(ei_spatial[1], N_POI)

    hc, hc_s = _transform(x_checkin, Wpc, bpc, deg_seq, N_CHECKIN)
    hp, hp_s = _transform(x_poi, Wpp, bpp, deg_sp, N_POI)

    agg_seq = _segsum(hc_s, ei_seq[0], ei_seq[1], N_CHECKIN)
    agg_vtd = _segsum(hp, ei_visited[0], ei_visited[1], N_CHECKIN)
    agg_vis = _segsum(hc, ei_visits[0], ei_visits[1], N_POI)
    agg_sp = _segsum(hp_s, ei_spatial[0], ei_spatial[1], N_POI)

    c1, c1_s = _combine(agg_seq, agg_vtd, hc, deg_seq, cnt_vtd,
                        l1_seq_W, l1_vtd_Wl, l1_vtd_Wr, l1_seq_b, l1_vtd_bl,
                        pa, N_CHECKIN, True, True)
    p1, p1_s = _combine(agg_sp, agg_vis, hp, deg_sp, cnt_vis,
                        l1_sp_W, l1_vis_Wl, l1_vis_Wr, l1_sp_b, l1_vis_bl,
                        pa, N_POI, True, True)

    agg_seq2 = _segsum(c1_s, ei_seq[0], ei_seq[1], N_CHECKIN)
    agg_vtd2 = _segsum(p1, ei_visited[0], ei_visited[1], N_CHECKIN)
    agg_vis2 = _segsum(c1, ei_visits[0], ei_visits[1], N_POI)
    agg_sp2 = _segsum(p1_s, ei_spatial[0], ei_spatial[1], N_POI)

    c2, _ = _combine(agg_seq2, agg_vtd2, c1, deg_seq, cnt_vtd,
                     l2_seq_W, l2_vtd_Wl, l2_vtd_Wr, l2_seq_b, l2_vtd_bl,
                     pa, N_CHECKIN, False, False)
    p2, _ = _combine(agg_sp2, agg_vis2, p1, deg_sp, cnt_vis,
                     l2_sp_W, l2_vis_Wl, l2_vis_Wr, l2_sp_b, l2_vis_bl,
                     pa, N_POI, False, False)
    return (c2, p2)
